# Initial kernel scaffold; baseline (speedup 1.0000x reference)
import jax
import jax.numpy as jnp
from jax.experimental import pallas as pl


def kernel(x, edge_index, l0_W1, l0_b1, l0_g1, l0_be1, l0_W2, l0_b2, l0_g2, l0_be2, l1_W1, l1_b1, l1_g1, l1_be1, l1_W2, l1_b2, l1_g2, l1_be2):
    raise NotImplementedError("write your pallas kernel here")


# speedup vs baseline: 7.0231x; 7.0231x over previous
import functools

import jax
import jax.numpy as jnp
from jax import lax
from jax.experimental import pallas as pl
from jax.experimental.pallas import tpu as pltpu
from jax.experimental.pallas import tpu_sc as plsc

_N, _E, _D = 10000, 320000, 128
_NC, _NS = 2, 16
_NW = _NC * _NS
_EPT = _E // _NW
_CH = 100
_NCH = _EPT // _CH
_STRIPE = 624
_REM_OFF = _STRIPE * _NS
_REM = _N - _REM_OFF


def _sc_scatter_body(x_hbm, src_hbm, dst_hbm, zero_hbm, out_hbm,
                     src_v, dst_v, rows_v, acc_sh, sem):
    c = lax.axis_index("c")
    s = lax.axis_index("s")
    wid = c * _NS + s
    pltpu.sync_copy(zero_hbm.at[pl.ds(s * _STRIPE, _STRIPE)],
                    acc_sh.at[pl.ds(s * _STRIPE, _STRIPE)])
    @pl.when(s == _NS - 1)
    def _():
        pltpu.sync_copy(zero_hbm.at[pl.ds(_REM_OFF, _REM)],
                        acc_sh.at[pl.ds(_REM_OFF, _REM)])
    pltpu.sync_copy(src_hbm.at[wid], src_v)
    pltpu.sync_copy(dst_hbm.at[wid], dst_v)
    plsc.subcore_barrier()

    def body(j, carry):
        pltpu.async_copy(x_hbm.at[src_v.at[j]], rows_v, sem).wait()
        pltpu.sync_copy(rows_v, acc_sh.at[dst_v.at[j]], add=True)
        return carry

    lax.fori_loop(0, _NCH, body, 0)
    plsc.subcore_barrier()
    pltpu.sync_copy(acc_sh.at[pl.ds(s * _STRIPE, _STRIPE)],
                    out_hbm.at[pl.ds(c * _N + s * _STRIPE, _STRIPE)])
    @pl.when(s == _NS - 1)
    def _():
        pltpu.sync_copy(acc_sh.at[pl.ds(_REM_OFF, _REM)],
                        out_hbm.at[pl.ds(c * _N + _REM_OFF, _REM)])


_sc_scatter = pl.kernel(
    _sc_scatter_body,
    out_type=jax.ShapeDtypeStruct((_NC * _N, _D), jnp.float32),
    mesh=plsc.VectorSubcoreMesh(core_axis_name="c", subcore_axis_name="s"),
    scratch_types=[
        pltpu.VMEM((_NCH, _CH), jnp.int32),
        pltpu.VMEM((_NCH, _CH), jnp.int32),
        pltpu.VMEM((_CH, _D), jnp.float32),
        pltpu.VMEM_SHARED((_N, _D), jnp.float32),
        pltpu.SemaphoreType.DMA,
    ],
)


def _mlp_body(h_ref, p_ref, x_ref, w1t_ref, b1_ref, g1_ref, be1_ref,
              w2t_ref, b2_ref, g2_ref, be2_ref, o_ref):
    z = h_ref[...] + p_ref[0:_N, :] + p_ref[_N:2 * _N, :]
    t = jnp.dot(z, w1t_ref[...], preferred_element_type=jnp.float32)
    t = t + b1_ref[...]
    m = jnp.mean(t, axis=0, keepdims=True)
    v = jnp.mean((t - m) * (t - m), axis=0, keepdims=True)
    t = (t - m) / jnp.sqrt(v + 1e-5) * g1_ref[...] + be1_ref[...]
    t = jnp.maximum(t, 0.0)
    u = jnp.dot(t, w2t_ref[...], preferred_element_type=jnp.float32)
    u = u + b2_ref[...]
    m2 = jnp.mean(u, axis=0, keepdims=True)
    v2 = jnp.mean((u - m2) * (u - m2), axis=0, keepdims=True)
    u = (u - m2) / jnp.sqrt(v2 + 1e-5) * g2_ref[...] + be2_ref[...]
    o_ref[...] = jnp.maximum(u, 0.0) + x_ref[...]


_mlp_call = pl.pallas_call(
    _mlp_body,
    out_shape=jax.ShapeDtypeStruct((_N, _D), jnp.float32),
)


def kernel(x, edge_index,
           l0_W1, l0_b1, l0_g1, l0_be1, l0_W2, l0_b2, l0_g2, l0_be2,
           l1_W1, l1_b1, l1_g1, l1_be1, l1_W2, l1_b2, l1_g2, l1_be2):
    src = edge_index[0].astype(jnp.int32).reshape(_NW, _NCH, _CH)
    dst = edge_index[1].astype(jnp.int32).reshape(_NW, _NCH, _CH)
    zero = jnp.zeros((_N, _D), jnp.float32)

    def layer(h, W1, b1, g1, be1, W2, b2, g2, be2):
        p = _sc_scatter(h, src, dst, zero)
        return _mlp_call(h, p, x,
                         W1.T, b1.reshape(1, _D), g1.reshape(1, _D),
                         be1.reshape(1, _D),
                         W2.T, b2.reshape(1, _D), g2.reshape(1, _D),
                         be2.reshape(1, _D))

    h = layer(x, l0_W1, l0_b1, l0_g1, l0_be1, l0_W2, l0_b2, l0_g2, l0_be2)
    return layer(h, l1_W1, l1_b1, l1_g1, l1_be1, l1_W2, l1_b2, l1_g2, l1_be2)
